# initial kernel scaffold (unmeasured)
import jax
import jax.numpy as jnp
from jax import lax
from jax.experimental import pallas as pl
from jax.experimental.pallas import tpu as pltpu

N_DEV = 16


def kernel(x, w_mat, scale_x, scale_w):
    m_full, k_shard = x.shape
    _, n = w_mat.shape
    m_blk = m_full // N_DEV

    def body(x_ref, w_ref, sx_ref, sw_ref, out_ref, comm_ref, send_sems,
             recv_sems):
        my = lax.axis_index("i")

        rdmas = []
        for d in range(1, N_DEV):
            tgt = (my + d) % N_DEV
            rdma = pltpu.make_async_remote_copy(
                src_ref=x_ref.at[pl.ds(tgt * m_blk, m_blk), :],
                dst_ref=comm_ref.at[d],
                send_sem=send_sems.at[d],
                recv_sem=recv_sems.at[d],
                device_id=(tgt,),
                device_id_type=pl.DeviceIdType.MESH,
            )
            rdma.start()
            rdmas.append(rdma)

        out_ref[...] = jnp.dot(
            x_ref[pl.ds(my * m_blk, m_blk), :],
            w_ref[pl.ds(my * k_shard, k_shard), :],
            preferred_element_type=jnp.float32,
        )

        for d in range(1, N_DEV):
            src = (my + N_DEV - d) % N_DEV
            rdmas[d - 1].wait_recv()
            out_ref[...] += jnp.dot(
                comm_ref[d],
                w_ref[pl.ds(src * k_shard, k_shard), :],
                preferred_element_type=jnp.float32,
            )

        for r in rdmas:
            r.wait_send()

        s = sx_ref[0] * sw_ref[0]
        out_ref[...] = jnp.maximum(out_ref[...] * s, 0.0)

    return pl.pallas_call(
        body,
        out_shape=jax.ShapeDtypeStruct((m_blk, n), jnp.float32),
        in_specs=[
            pl.BlockSpec(memory_space=pltpu.VMEM),
            pl.BlockSpec(memory_space=pltpu.VMEM),
            pl.BlockSpec(memory_space=pltpu.SMEM),
            pl.BlockSpec(memory_space=pltpu.SMEM),
        ],
        out_specs=pl.BlockSpec(memory_space=pltpu.VMEM),
        scratch_shapes=[
            pltpu.VMEM((N_DEV, m_blk, k_shard), x.dtype),
            pltpu.SemaphoreType.DMA((N_DEV,)),
            pltpu.SemaphoreType.DMA((N_DEV,)),
        ],
        compiler_params=pltpu.CompilerParams(collective_id=0),
    )(x, w_mat, scale_x, scale_w)


# baseline (device time: 58197 ns/iter reference)
import jax
import jax.numpy as jnp
from jax import lax
from jax.experimental import pallas as pl
from jax.experimental.pallas import tpu as pltpu

N_DEV = 16


def kernel(x, w_mat, scale_x, scale_w):
    m_full, k_shard = x.shape
    _, n = w_mat.shape
    m_blk = m_full // N_DEV

    def body(x_ref, w_hbm, sx_ref, sw_ref, out_ref, stage_ref, comm_ref,
             w_buf, send_sems, recv_sems, w_sems):
        my = lax.axis_index("i")

        stage_ref[...] = x_ref[...].reshape(N_DEV, m_blk, k_shard).astype(
            jnp.float8_e5m2)

        rdmas = []
        for d in range(1, N_DEV):
            tgt = (my + d) % N_DEV
            rdma = pltpu.make_async_remote_copy(
                src_ref=stage_ref.at[tgt],
                dst_ref=comm_ref.at[d],
                send_sem=send_sems.at[d],
                recv_sem=recv_sems.at[d],
                device_id=(tgt,),
                device_id_type=pl.DeviceIdType.MESH,
            )
            rdma.start()
            rdmas.append(rdma)

        srcs = [my] + [(my + N_DEV - d) % N_DEV for d in range(1, N_DEV)]

        def load_w(slot, src):
            cp = pltpu.make_async_copy(
                w_hbm.at[pl.ds(src * k_shard, k_shard), :],
                w_buf.at[slot],
                w_sems.at[slot],
            )
            cp.start()
            return cp

        w_cps = [load_w(0, srcs[0])]
        for d in range(N_DEV):
            if d + 1 < N_DEV:
                w_cps.append(load_w((d + 1) % 2, srcs[d + 1]))
            w_cps[d].wait()
            if d == 0:
                blk = stage_ref[my]
            else:
                rdmas[d - 1].wait_recv()
                blk = comm_ref[d]
            part = jnp.dot(
                blk.astype(jnp.bfloat16),
                w_buf[d % 2].astype(jnp.bfloat16),
                preferred_element_type=jnp.float32,
            )
            if d == 0:
                out_ref[...] = part
            else:
                out_ref[...] += part

        for r in rdmas:
            r.wait_send()

        s = sx_ref[0] * sw_ref[0]
        out_ref[...] = jnp.maximum(out_ref[...] * s, 0.0)

    return pl.pallas_call(
        body,
        out_shape=jax.ShapeDtypeStruct((m_blk, n), jnp.float32),
        in_specs=[
            pl.BlockSpec(memory_space=pltpu.VMEM),
            pl.BlockSpec(memory_space=pl.ANY),
            pl.BlockSpec(memory_space=pltpu.SMEM),
            pl.BlockSpec(memory_space=pltpu.SMEM),
        ],
        out_specs=pl.BlockSpec(memory_space=pltpu.VMEM),
        scratch_shapes=[
            pltpu.VMEM((N_DEV, m_blk, k_shard), jnp.float8_e5m2),
            pltpu.VMEM((N_DEV, m_blk, k_shard), jnp.float8_e5m2),
            pltpu.VMEM((2, k_shard, n), jnp.float32),
            pltpu.SemaphoreType.DMA((N_DEV,)),
            pltpu.SemaphoreType.DMA((N_DEV,)),
            pltpu.SemaphoreType.DMA((2,)),
        ],
    )(x, w_mat, scale_x, scale_w)


# device time: 58137 ns/iter; 1.0010x vs baseline; 1.0010x over previous
import jax
import jax.numpy as jnp
from jax import lax
from jax.experimental import pallas as pl
from jax.experimental.pallas import tpu as pltpu

N_DEV = 16


def kernel(x, w_mat, scale_x, scale_w):
    m_full, k_shard = x.shape
    _, n = w_mat.shape
    m_blk = m_full // N_DEV

    def body(x_ref, w_hbm, sx_ref, sw_ref, out_ref, stage_ref, comm_ref,
             w_buf, send_sems, recv_sems, w_sems):
        my = lax.axis_index("i")

        stage_ref[...] = x_ref[...].reshape(N_DEV, m_blk, k_shard).astype(
            jnp.float8_e5m2)

        rdmas = []
        for d in range(1, N_DEV):
            tgt = (my + d) % N_DEV
            rdma = pltpu.make_async_remote_copy(
                src_ref=stage_ref.at[tgt],
                dst_ref=comm_ref.at[d],
                send_sem=send_sems.at[d],
                recv_sem=recv_sems.at[d],
                device_id=(tgt,),
                device_id_type=pl.DeviceIdType.MESH,
            )
            rdma.start()
            rdmas.append(rdma)

        srcs = [my] + [(my + N_DEV - d) % N_DEV for d in range(1, N_DEV)]

        def load_w(slot, src):
            cp = pltpu.make_async_copy(
                w_hbm.at[pl.ds(src * k_shard, k_shard), :],
                w_buf.at[slot],
                w_sems.at[slot],
            )
            cp.start()
            return cp

        w_cps = [load_w(0, srcs[0])]
        for d in range(N_DEV):
            if d + 1 < N_DEV:
                w_cps.append(load_w((d + 1) % 2, srcs[d + 1]))
            w_cps[d].wait()
            if d == 0:
                blk = stage_ref[my]
            else:
                rdmas[d - 1].wait_recv()
                blk = comm_ref[d]
            part = jnp.dot(
                blk,
                w_buf[d % 2].astype(jnp.float8_e5m2),
                preferred_element_type=jnp.float32,
            )
            if d == 0:
                out_ref[...] = part
            else:
                out_ref[...] += part

        for r in rdmas:
            r.wait_send()

        s = sx_ref[0] * sw_ref[0]
        out_ref[...] = jnp.maximum(out_ref[...] * s, 0.0)

    return pl.pallas_call(
        body,
        out_shape=jax.ShapeDtypeStruct((m_blk, n), jnp.float32),
        in_specs=[
            pl.BlockSpec(memory_space=pltpu.VMEM),
            pl.BlockSpec(memory_space=pl.ANY),
            pl.BlockSpec(memory_space=pltpu.SMEM),
            pl.BlockSpec(memory_space=pltpu.SMEM),
        ],
        out_specs=pl.BlockSpec(memory_space=pltpu.VMEM),
        scratch_shapes=[
            pltpu.VMEM((N_DEV, m_blk, k_shard), jnp.float8_e5m2),
            pltpu.VMEM((N_DEV, m_blk, k_shard), jnp.float8_e5m2),
            pltpu.VMEM((2, k_shard, n), jnp.float32),
            pltpu.SemaphoreType.DMA((N_DEV,)),
            pltpu.SemaphoreType.DMA((N_DEV,)),
            pltpu.SemaphoreType.DMA((2,)),
        ],
    )(x, w_mat, scale_x, scale_w)
